# baseline (speedup 1.0000x reference)
import functools

import jax
import jax.numpy as jnp
from jax import lax
from jax.experimental import pallas as pl
from jax.experimental.pallas import tpu as pltpu
from jax.experimental.pallas import tpu_sc as plsc

B = 16384
D = 128
NTOK = 10
NB = 10
NK = D // 16

_NC = 2
_NS = 16
_NW = _NC * _NS
_CPW = B // _NW
_CH = 8
_CH2 = 32

_OFF_CLS = 0
_OFF_WM = 1
_OFF_WP = 2
_OFF_EMB = (3, 6, 9, 12, 15, 25)
_OFF_BB = 35
_OFF_WB = 36
_OFF_BM = 46
_OFF_BP = 47
_TROWS = 48


def _sl(k):
    return pl.ds(k * 16, 16)


def _body(cls_hbm, wm_hbm, wp_hbm, es1_hbm, es2_hbm, eo1_hbm, eo2_hbm,
          eg1_hbm, eg2_hbm, bb_hbm, wb_hbm, bmi_hbm, bpc_hbm,
          s1_hbm, s2_hbm, o1_hbm, o2_hbm, g1_hbm, g2_hbm,
          miss_hbm, pc_hbm, bool_hbm, out_hbm,
          tv, idxv, scal, boolv, clsb,
          ob1, ob2, ob3, ob4, ob5, ob6, ob7, ob8, ob9, sem0):
    wid = lax.axis_index("s") * _NC + lax.axis_index("c")
    base = wid * _CPW
    obt = (ob1, ob2, ob3, ob4, ob5, ob6, ob7, ob8, ob9)

    stage = [
        (cls_hbm, tv.at[pl.ds(_OFF_CLS, 1)]),
        (wm_hbm, tv.at[pl.ds(_OFF_WM, 1)]),
        (wp_hbm, tv.at[pl.ds(_OFF_WP, 1)]),
        (es1_hbm, tv.at[pl.ds(_OFF_EMB[0], 3)]),
        (es2_hbm, tv.at[pl.ds(_OFF_EMB[1], 3)]),
        (eo1_hbm, tv.at[pl.ds(_OFF_EMB[2], 3)]),
        (eo2_hbm, tv.at[pl.ds(_OFF_EMB[3], 3)]),
        (eg1_hbm, tv.at[pl.ds(_OFF_EMB[4], 10)]),
        (eg2_hbm, tv.at[pl.ds(_OFF_EMB[5], 10)]),
        (bb_hbm, tv.at[pl.ds(_OFF_BB, 1)]),
        (wb_hbm, tv.at[pl.ds(_OFF_WB, 10)]),
        (bmi_hbm, tv.at[pl.ds(_OFF_BM, 1)]),
        (bpc_hbm, tv.at[pl.ds(_OFF_BP, 1)]),
        (s1_hbm.at[pl.ds(base, _CPW)], idxv.at[0]),
        (s2_hbm.at[pl.ds(base, _CPW)], idxv.at[1]),
        (o1_hbm.at[pl.ds(base, _CPW)], idxv.at[2]),
        (o2_hbm.at[pl.ds(base, _CPW)], idxv.at[3]),
        (g1_hbm.at[pl.ds(base, _CPW)], idxv.at[4]),
        (g2_hbm.at[pl.ds(base, _CPW)], idxv.at[5]),
        (miss_hbm.at[pl.ds(base, _CPW)], scal.at[0]),
        (pc_hbm.at[pl.ds(base, _CPW)], scal.at[1]),
        (bool_hbm.at[pl.ds(base, _CPW), :], boolv),
    ]
    copies = [pltpu.async_copy(src, dst, sem0) for src, dst in stage]
    for cp in copies:
        cp.wait()

    for k in range(NK):
        v = tv[_OFF_CLS, _sl(k)]
        for l in range(_CH2):
            clsb[l, _sl(k)] = v

    def chunk_body(c, carry):
        r0 = c * _CH2
        gbase = base + r0
        sem = sem0

        @pl.when(c > 0)
        def _reclaim():
            pltpu.make_async_copy(
                clsb, out_hbm.at[pl.ds(0, _CH2), 0], sem).wait()
            for t in range(1, NTOK):
                pltpu.make_async_copy(
                    obt[t - 1], out_hbm.at[pl.ds(0, _CH2), t], sem).wait()

        wm = [tv[_OFF_WM, _sl(k)] for k in range(NK)]
        bm = [tv[_OFF_BM, _sl(k)] for k in range(NK)]
        wp = [tv[_OFF_WP, _sl(k)] for k in range(NK)]
        bp = [tv[_OFF_BP, _sl(k)] for k in range(NK)]

        @plsc.parallel_loop(0, _CH2, unroll=2)
        def row_a(row):
            rs = jnp.full((16,), r0, jnp.int32) + row
            mg = plsc.load_gather(scal, [jnp.full((16,), 0, jnp.int32), rs])
            pg = plsc.load_gather(scal, [jnp.full((16,), 1, jnp.int32), rs])
            tix = [plsc.load_gather(
                idxv, [jnp.full((16,), t, jnp.int32), rs])[0] + _OFF_EMB[t]
                for t in range(6)]
            for k in range(NK):
                sl = _sl(k)
                ob1[row, sl] = wm[k] * mg + bm[k]
                ob2[row, sl] = wp[k] * pg + bp[k]
                for t in range(6):
                    obt[t + 2][row, sl] = tv[tix[t], sl]

        pltpu.async_copy(clsb, out_hbm.at[pl.ds(gbase, _CH2), 0], sem)
        for t in range(1, 9):
            pltpu.async_copy(
                obt[t - 1], out_hbm.at[pl.ds(gbase, _CH2), t], sem)

        for kh in range(2):
            wb = [[tv[_OFF_WB + j, _sl(kh * 4 + k)] for j in range(NB)]
                  for k in range(4)]
            bb = [tv[_OFF_BB, _sl(kh * 4 + k)] for k in range(4)]

            @plsc.parallel_loop(0, _CH2, unroll=1)
            def row_b(row):
                rs = jnp.full((16,), r0, jnp.int32) + row
                bg = [plsc.load_gather(
                    boolv, [rs, jnp.full((16,), j, jnp.int32)])
                    for j in range(NB)]
                for k in range(4):
                    ps = [wb[k][j] * bg[j] for j in range(NB)]
                    while len(ps) > 1:
                        nxt = [ps[i] + ps[i + 1]
                               for i in range(0, len(ps) - 1, 2)]
                        if len(ps) % 2:
                            nxt.append(ps[-1])
                        ps = nxt
                    ob9[row, _sl(kh * 4 + k)] = ps[0] + bb[k]

        pltpu.async_copy(obt[8], out_hbm.at[pl.ds(gbase, _CH2), 9], sem)
        return carry

    lax.fori_loop(0, _CPW // _CH2, chunk_body, 0)

    pltpu.make_async_copy(
        clsb, out_hbm.at[pl.ds(0, _CH2), 0], sem0).wait()
    for t in range(1, NTOK):
        pltpu.make_async_copy(
            obt[t - 1], out_hbm.at[pl.ds(0, _CH2), t], sem0).wait()


@jax.jit
def _run(cls2d, W_miss, W_pc, E_sat1, E_sat2, E_obj1, E_obj2, E_org1,
         E_org2, bb2d, W_bool, bm2d, bp2d,
         s1, s2, o1, o2, g1, g2, miss, pc, bools):
    call = functools.partial(
        pl.kernel,
        out_type=jax.ShapeDtypeStruct((B, NTOK, D), jnp.float32),
        mesh=plsc.VectorSubcoreMesh(core_axis_name="c", subcore_axis_name="s"),
        compiler_params=pltpu.CompilerParams(needs_layout_passes=False),
        scratch_types=(
            [pltpu.VMEM((_TROWS, D), jnp.float32),
             pltpu.VMEM((6, _CPW), jnp.int32),
             pltpu.VMEM((2, _CPW), jnp.float32),
             pltpu.VMEM((_CPW, NB), jnp.float32),
             pltpu.VMEM((_CH2, D), jnp.float32)]
            + [pltpu.VMEM((_CH2, D), jnp.float32) for _ in range(9)]
            + [pltpu.SemaphoreType.DMA]
        ),
    )(_body)
    return call(cls2d, W_miss, W_pc, E_sat1, E_sat2, E_obj1, E_obj2,
                E_org1, E_org2, bb2d, W_bool, bm2d, bp2d,
                s1, s2, o1, o2, g1, g2, miss, pc, bools)


def kernel(miss_distance, pc, sat1_type, sat2_type, obj1_type, obj2_type,
           org1, org2, bool_features, W_miss, b_miss, W_pc, b_pc,
           E_sat1, E_sat2, E_obj1, E_obj2, E_org1, E_org2, W_bool, b_bool,
           CLS):
    return _run(
        CLS.reshape(1, D), W_miss, W_pc,
        E_sat1, E_sat2, E_obj1, E_obj2, E_org1, E_org2,
        b_bool.reshape(1, D), W_bool,
        b_miss.reshape(1, D), b_pc.reshape(1, D),
        sat1_type.astype(jnp.int32), sat2_type.astype(jnp.int32),
        obj1_type.astype(jnp.int32), obj2_type.astype(jnp.int32),
        org1.astype(jnp.int32), org2.astype(jnp.int32),
        miss_distance.reshape(B), pc.reshape(B), bool_features)


# speedup vs baseline: 1.3509x; 1.3509x over previous
import functools

import jax
import jax.numpy as jnp
from jax import lax
from jax.experimental import pallas as pl
from jax.experimental.pallas import tpu as pltpu
from jax.experimental.pallas import tpu_sc as plsc

B = 16384
D = 128
NTOK = 10
NB = 10
NK = D // 16

_NC = 2
_NS = 16
_NW = _NC * _NS
_CPW = B // _NW
_CH = 8
_CH2 = 32

_OFF_CLS = 0
_OFF_WM = 1
_OFF_WP = 2
_OFF_EMB = (3, 6, 9, 12, 15, 25)
_OFF_BB = 35
_OFF_WB = 36
_OFF_BM = 46
_OFF_BP = 47
_TROWS = 48


def _sl(k):
    return pl.ds(k * 16, 16)


def _body(cls_hbm, wm_hbm, wp_hbm, es1_hbm, es2_hbm, eo1_hbm, eo2_hbm,
          eg1_hbm, eg2_hbm, bb_hbm, wb_hbm, bmi_hbm, bpc_hbm,
          s1_hbm, s2_hbm, o1_hbm, o2_hbm, g1_hbm, g2_hbm,
          miss_hbm, pc_hbm, bool_hbm, out_hbm,
          tv, idxv, scal, boolv, clsb,
          ob1, ob2, ob3, ob4, ob5, ob6, ob7, ob8, ob9, sem0):
    wid = lax.axis_index("s") * _NC + lax.axis_index("c")
    base = wid * _CPW
    obt = (ob1, ob2, ob3, ob4, ob5, ob6, ob7, ob8, ob9)

    stage = [
        (cls_hbm, tv.at[pl.ds(_OFF_CLS, 1)]),
        (wm_hbm, tv.at[pl.ds(_OFF_WM, 1)]),
        (wp_hbm, tv.at[pl.ds(_OFF_WP, 1)]),
        (es1_hbm, tv.at[pl.ds(_OFF_EMB[0], 3)]),
        (es2_hbm, tv.at[pl.ds(_OFF_EMB[1], 3)]),
        (eo1_hbm, tv.at[pl.ds(_OFF_EMB[2], 3)]),
        (eo2_hbm, tv.at[pl.ds(_OFF_EMB[3], 3)]),
        (eg1_hbm, tv.at[pl.ds(_OFF_EMB[4], 10)]),
        (eg2_hbm, tv.at[pl.ds(_OFF_EMB[5], 10)]),
        (bb_hbm, tv.at[pl.ds(_OFF_BB, 1)]),
        (wb_hbm, tv.at[pl.ds(_OFF_WB, 10)]),
        (bmi_hbm, tv.at[pl.ds(_OFF_BM, 1)]),
        (bpc_hbm, tv.at[pl.ds(_OFF_BP, 1)]),
        (s1_hbm.at[pl.ds(base, _CPW)], idxv.at[0]),
        (s2_hbm.at[pl.ds(base, _CPW)], idxv.at[1]),
        (o1_hbm.at[pl.ds(base, _CPW)], idxv.at[2]),
        (o2_hbm.at[pl.ds(base, _CPW)], idxv.at[3]),
        (g1_hbm.at[pl.ds(base, _CPW)], idxv.at[4]),
        (g2_hbm.at[pl.ds(base, _CPW)], idxv.at[5]),
        (miss_hbm.at[pl.ds(base, _CPW)], scal.at[0]),
        (pc_hbm.at[pl.ds(base, _CPW)], scal.at[1]),
        (bool_hbm.at[pl.ds(base, _CPW), :], boolv),
    ]
    copies = [pltpu.async_copy(src, dst, sem0) for src, dst in stage]
    for cp in copies:
        cp.wait()

    for k in range(NK):
        v = tv[_OFF_CLS, _sl(k)]
        for l in range(_CH2):
            clsb[l, _sl(k)] = v

    def chunk_body(c, carry):
        r0 = c * _CH2
        gbase = base + r0
        sem = sem0

        @pl.when(c > 0)
        def _reclaim():
            pltpu.make_async_copy(
                clsb, out_hbm.at[pl.ds(0, _CH2), 0], sem).wait()
            for t in range(1, NTOK):
                pltpu.make_async_copy(
                    obt[t - 1], out_hbm.at[pl.ds(0, _CH2), t], sem).wait()

        wm = [tv[_OFF_WM, _sl(k)] for k in range(NK)]
        bm = [tv[_OFF_BM, _sl(k)] for k in range(NK)]
        wp = [tv[_OFF_WP, _sl(k)] for k in range(NK)]
        bp = [tv[_OFF_BP, _sl(k)] for k in range(NK)]

        @plsc.parallel_loop(0, _CH2, unroll=1)
        def row_a(row):
            rs = jnp.full((16,), r0, jnp.int32) + row
            mg = plsc.load_gather(scal, [jnp.full((16,), 0, jnp.int32), rs])
            pg = plsc.load_gather(scal, [jnp.full((16,), 1, jnp.int32), rs])
            tix = [plsc.load_gather(
                idxv, [jnp.full((16,), t, jnp.int32), rs])[0] + _OFF_EMB[t]
                for t in range(6)]
            for k in range(NK):
                sl = _sl(k)
                ob1[row, sl] = wm[k] * mg + bm[k]
                ob2[row, sl] = wp[k] * pg + bp[k]
                for t in range(6):
                    obt[t + 2][row, sl] = tv[tix[t], sl]

        pltpu.async_copy(clsb, out_hbm.at[pl.ds(gbase, _CH2), 0], sem)
        for t in range(1, 9):
            pltpu.async_copy(
                obt[t - 1], out_hbm.at[pl.ds(gbase, _CH2), t], sem)

        for kh in range(2):
            wb = [[tv[_OFF_WB + j, _sl(kh * 4 + k)] for j in range(NB)]
                  for k in range(4)]
            bb = [tv[_OFF_BB, _sl(kh * 4 + k)] for k in range(4)]

            @plsc.parallel_loop(0, _CH2, unroll=1)
            def row_b(row):
                rs = jnp.full((16,), r0, jnp.int32) + row
                bg = [plsc.load_gather(
                    boolv, [rs, jnp.full((16,), j, jnp.int32)])
                    for j in range(NB)]
                for k in range(4):
                    ps = [wb[k][j] * bg[j] for j in range(NB)]
                    while len(ps) > 1:
                        nxt = [ps[i] + ps[i + 1]
                               for i in range(0, len(ps) - 1, 2)]
                        if len(ps) % 2:
                            nxt.append(ps[-1])
                        ps = nxt
                    ob9[row, _sl(kh * 4 + k)] = ps[0] + bb[k]

        pltpu.async_copy(obt[8], out_hbm.at[pl.ds(gbase, _CH2), 9], sem)
        return carry

    lax.fori_loop(0, _CPW // _CH2, chunk_body, 0)

    pltpu.make_async_copy(
        clsb, out_hbm.at[pl.ds(0, _CH2), 0], sem0).wait()
    for t in range(1, NTOK):
        pltpu.make_async_copy(
            obt[t - 1], out_hbm.at[pl.ds(0, _CH2), t], sem0).wait()


@jax.jit
def _run(cls2d, W_miss, W_pc, E_sat1, E_sat2, E_obj1, E_obj2, E_org1,
         E_org2, bb2d, W_bool, bm2d, bp2d,
         s1, s2, o1, o2, g1, g2, miss, pc, bools):
    call = functools.partial(
        pl.kernel,
        out_type=jax.ShapeDtypeStruct((B, NTOK, D), jnp.float32),
        mesh=plsc.VectorSubcoreMesh(core_axis_name="c", subcore_axis_name="s"),
        compiler_params=pltpu.CompilerParams(needs_layout_passes=False),
        scratch_types=(
            [pltpu.VMEM((_TROWS, D), jnp.float32),
             pltpu.VMEM((6, _CPW), jnp.int32),
             pltpu.VMEM((2, _CPW), jnp.float32),
             pltpu.VMEM((_CPW, NB), jnp.float32),
             pltpu.VMEM((_CH2, D), jnp.float32)]
            + [pltpu.VMEM((_CH2, D), jnp.float32) for _ in range(9)]
            + [pltpu.SemaphoreType.DMA]
        ),
    )(_body)
    return call(cls2d, W_miss, W_pc, E_sat1, E_sat2, E_obj1, E_obj2,
                E_org1, E_org2, bb2d, W_bool, bm2d, bp2d,
                s1, s2, o1, o2, g1, g2, miss, pc, bools)


def kernel(miss_distance, pc, sat1_type, sat2_type, obj1_type, obj2_type,
           org1, org2, bool_features, W_miss, b_miss, W_pc, b_pc,
           E_sat1, E_sat2, E_obj1, E_obj2, E_org1, E_org2, W_bool, b_bool,
           CLS):
    return _run(
        CLS.reshape(1, D), W_miss, W_pc,
        E_sat1, E_sat2, E_obj1, E_obj2, E_org1, E_org2,
        b_bool.reshape(1, D), W_bool,
        b_miss.reshape(1, D), b_pc.reshape(1, D),
        sat1_type.astype(jnp.int32), sat2_type.astype(jnp.int32),
        obj1_type.astype(jnp.int32), obj2_type.astype(jnp.int32),
        org1.astype(jnp.int32), org2.astype(jnp.int32),
        miss_distance.reshape(B), pc.reshape(B), bool_features)
